# baseline (speedup 1.0000x reference)
import functools

import jax
import jax.numpy as jnp
from jax import lax
from jax.experimental import pallas as pl
from jax.experimental.pallas import tpu as pltpu
from jax.experimental.pallas import tpu_sc as plsc

N = 10000
E = 320000
D = 128
DH = D // 2

NC = 2
NS = 16
NW = NC * NS
C = 125
CH = E // (NW * C)
NPAD = 10240
ZB = NPAD // NS


def _make_sc_agg():
    mesh = plsc.VectorSubcoreMesh(
        core_axis_name="c", subcore_axis_name="s",
        num_cores=NC, num_subcores=NS)
    return functools.partial(
        pl.kernel,
        out_type=jax.ShapeDtypeStruct((NC, 2, NPAD, DH), jnp.float32),
        mesh=mesh,
        scratch_types=[
            pltpu.VMEM((CH, C), jnp.int32),
            pltpu.VMEM((CH, C), jnp.int32),
            pltpu.VMEM((CH * C,), jnp.float32),
            pltpu.VMEM((C, DH), jnp.float32),
            pltpu.VMEM((C, DH), jnp.float32),
            pltpu.VMEM((64, DH), jnp.float32),
            pltpu.VMEM_SHARED((NPAD, DH), jnp.float32),
            pltpu.SemaphoreType.DMA,
            pltpu.SemaphoreType.DMA,
        ],
        compiler_params=pltpu.CompilerParams(
            needs_layout_passes=False, use_tc_tiling_on_sc=False),
    )(_sc_agg_body)


def _sc_agg_body(y0_hbm, y1_hbm, row_hbm, col_hbm, ev_hbm, out_hbm,
                 rowv, colv, evv, bufa, bufb, zbuf, acc, sema, semb):
    cid = lax.axis_index("c")
    sid = lax.axis_index("s")
    wid = sid * NC + cid
    pltpu.sync_copy(row_hbm.at[wid], rowv)
    pltpu.sync_copy(col_hbm.at[wid], colv)
    pltpu.sync_copy(ev_hbm.at[wid], evv)

    zero = jnp.zeros((16,), jnp.float32)

    def zrow(i, _):
        for j in range(DH // 16):
            zbuf[i, pl.ds(j * 16, 16)] = zero
        return 0

    lax.fori_loop(0, 64, zrow, 0)

    def scale_scatter(ch, buf):
        def body(i, _):
            sc = plsc.load_gather(evv, [jnp.full((16,), ch * C + i, jnp.int32)])
            for j in range(DH // 16):
                buf[i, pl.ds(j * 16, 16)] = buf[i, pl.ds(j * 16, 16)] * sc
            return 0

        lax.fori_loop(0, C, body, 0)
        pltpu.sync_copy(buf, acc.at[colv.at[ch]], add=True)

    for half, y_hbm in ((0, y0_hbm), (1, y1_hbm)):
        for k in range(ZB // 64):
            pltpu.sync_copy(zbuf, acc.at[pl.ds(sid * ZB + k * 64, 64)])
        plsc.subcore_barrier()

        def gather(ch, buf, sem):
            pltpu.async_copy(y_hbm.at[rowv.at[ch]], buf, sem)

        def wait(buf, sem):
            pltpu.make_async_copy(y_hbm.at[rowv.at[0]], buf, sem).wait()

        gather(0, bufa, sema)

        def outer(t, _):
            ch = 2 * t
            gather(ch + 1, bufb, semb)
            wait(bufa, sema)
            scale_scatter(ch, bufa)

            @pl.when(ch + 2 < CH)
            def _():
                gather(ch + 2, bufa, sema)

            wait(bufb, semb)
            scale_scatter(ch + 1, bufb)
            return 0

        lax.fori_loop(0, CH // 2, outer, 0)
        plsc.subcore_barrier()
        pltpu.sync_copy(acc.at[pl.ds(sid * ZB, ZB)],
                        out_hbm.at[cid, half, pl.ds(sid * ZB, ZB)])
        plsc.subcore_barrier()


_SC_CACHE = {}


def _sc_agg(y0, y1, row3, col3, ev2):
    if "agg" not in _SC_CACHE:
        _SC_CACHE["agg"] = _make_sc_agg()
    return _SC_CACHE["agg"](y0, y1, row3, col3, ev2)


_MB = 2000


def _mm_raw_body(x_ref, w_ref, o_ref):
    o_ref[...] = jnp.dot(x_ref[...], w_ref[...],
                         preferred_element_type=jnp.float32)


def _tc_mm_raw(x, w):
    return pl.pallas_call(
        _mm_raw_body,
        grid=(N // _MB,),
        in_specs=[
            pl.BlockSpec((_MB, D), lambda i: (i, 0)),
            pl.BlockSpec((D, D), lambda i: (0, 0)),
        ],
        out_specs=pl.BlockSpec((_MB, D), lambda i: (i, 0)),
        out_shape=jax.ShapeDtypeStruct((N, D), jnp.float32),
    )(x, w)


def _mm2_body(x_ref, w_ref, dis_ref, y0_ref, y1_ref):
    acc = jnp.dot(x_ref[...], w_ref[...], preferred_element_type=jnp.float32)
    y = dis_ref[...] * acc
    y0_ref[...] = y[:, :DH]
    y1_ref[...] = y[:, DH:]


def _tc_mm2(x, w, dis):
    return pl.pallas_call(
        _mm2_body,
        grid=(N // _MB,),
        in_specs=[
            pl.BlockSpec((_MB, D), lambda i: (i, 0)),
            pl.BlockSpec((D, D), lambda i: (0, 0)),
            pl.BlockSpec((_MB, 1), lambda i: (i, 0)),
        ],
        out_specs=[
            pl.BlockSpec((_MB, DH), lambda i: (i, 0)),
            pl.BlockSpec((_MB, DH), lambda i: (i, 0)),
        ],
        out_shape=[
            jax.ShapeDtypeStruct((N, DH), jnp.float32),
            jax.ShapeDtypeStruct((N, DH), jnp.float32),
        ],
    )(x, w, dis)


def _epi_body(a00_ref, a01_ref, a10_ref, a11_ref, y0_ref, y1_ref, dis_ref,
              h_ref):
    dis = dis_ref[...]
    h_ref[:, :DH] = jnp.maximum(
        dis * (a00_ref[...] + a10_ref[...] + y0_ref[...]), 0.0)
    h_ref[:, DH:] = jnp.maximum(
        dis * (a01_ref[...] + a11_ref[...] + y1_ref[...]), 0.0)


def _tc_epi(a, y0, y1, dis):
    return pl.pallas_call(
        _epi_body,
        grid=(N // _MB,),
        in_specs=[
            pl.BlockSpec((_MB, DH), lambda i: (i, 0)),
            pl.BlockSpec((_MB, DH), lambda i: (i, 0)),
            pl.BlockSpec((_MB, DH), lambda i: (i, 0)),
            pl.BlockSpec((_MB, DH), lambda i: (i, 0)),
            pl.BlockSpec((_MB, DH), lambda i: (i, 0)),
            pl.BlockSpec((_MB, DH), lambda i: (i, 0)),
            pl.BlockSpec((_MB, 1), lambda i: (i, 0)),
        ],
        out_specs=pl.BlockSpec((_MB, D), lambda i: (i, 0)),
        out_shape=jax.ShapeDtypeStruct((N, D), jnp.float32),
    )(a[0, 0], a[0, 1], a[1, 0], a[1, 1], y0, y1, dis)


def _gru_cell(xi, h, Wih, Whh, bih, bhh):
    gi = xi @ Wih.T + bih
    gh = h @ Whh.T + bhh
    ir, iz, inn = jnp.split(gi, 3, axis=-1)
    hr, hz, hn = jnp.split(gh, 3, axis=-1)
    r = jax.nn.sigmoid(ir + hr)
    z = jax.nn.sigmoid(iz + hz)
    n = jnp.tanh(inn + r * hn)
    return (1.0 - z) * n + z * h


def _layer_weight(xin, score, Wih, Whh, bih, bhh, Winit):
    vals, idx = jax.lax.top_k(score, D)
    xt = xin[idx] * vals[:, None]
    return _gru_cell(xt, Winit, Wih, Whh, bih, bhh)


def kernel(x, edge_index, edge_weight, p1, Wih1, Whh1, bih1, bhh1, Winit1,
           p2, Wih2, Whh2, bih2, bhh2, Winit2, lin1_W, lin1_b, lin2_W, lin2_b):
    row, col = edge_index[0], edge_index[1]
    keep = (row != col).astype(edge_weight.dtype)
    ev = edge_weight * keep

    loop = jnp.arange(N, dtype=row.dtype)
    row_f = jnp.concatenate([row, loop])
    col_f = jnp.concatenate([col, loop])
    ew_f = jnp.concatenate([ev, jnp.ones((N,), dtype=edge_weight.dtype)])
    deg = jax.ops.segment_sum(ew_f, col_f, num_segments=N)
    safe = jnp.where(deg > 0, deg, 1.0)
    dis = jnp.where(deg > 0, 1.0 / jnp.sqrt(safe), 0.0)
    norm = dis[row_f] * ew_f * dis[col_f]

    score1 = jnp.tanh((x @ p1) / (jnp.linalg.norm(p1) + 1e-16))
    W1 = _layer_weight(x, score1, Wih1, Whh1, bih1, bhh1, Winit1)
    xw1 = _tc_mm_raw(x, W1)
    h = jax.nn.relu(
        jax.ops.segment_sum(norm[:, None] * xw1[row_f], col_f, num_segments=N))

    score2 = jnp.tanh((h @ p2) / (jnp.linalg.norm(p2) + 1e-16))
    W2 = _layer_weight(h, score2, Wih2, Whh2, bih2, bhh2, Winit2)

    dis2d = dis[:, None]
    y20, y21 = _tc_mm2(h, W2, dis2d)

    row3 = row.reshape(NW, CH, C)
    col3 = col.reshape(NW, CH, C)
    ev2 = ev.reshape(NW, CH * C)
    a2 = _sc_agg(y20, y21, row3, col3, ev2)[:, :, :N]
    h2 = _tc_epi(a2, y20, y21, dis2d)

    h2 = h2 @ lin1_W.T + lin1_b
    h2 = h2 @ lin2_W.T + lin2_b
    return h2


# speedup vs baseline: 2.8364x; 2.8364x over previous
import functools

import jax
import jax.numpy as jnp
from jax import lax
from jax.experimental import pallas as pl
from jax.experimental.pallas import tpu as pltpu
from jax.experimental.pallas import tpu_sc as plsc

N = 10000
E = 320000
D = 128
DH = D // 2

NC = 2
NS = 16
NW = NC * NS
C = 125
CH = E // (NW * C)
NPAD = 10240
ZB = NPAD // NS


EF = E + N
CC = 120
WSLOT = 10320
EFPAD = NW * WSLOT


def _make_sc_contrib():
    mesh = plsc.VectorSubcoreMesh(
        core_axis_name="c", subcore_axis_name="s",
        num_cores=NC, num_subcores=NS)
    return functools.partial(
        pl.kernel,
        out_type=jax.ShapeDtypeStruct((EF, D), jnp.float32),
        mesh=mesh,
        scratch_types=[
            pltpu.VMEM((WSLOT,), jnp.int32),
            pltpu.VMEM((WSLOT,), jnp.int32),
            pltpu.VMEM((WSLOT,), jnp.float32),
            pltpu.VMEM((WSLOT,), jnp.float32),
            pltpu.VMEM((N,), jnp.float32),
            pltpu.VMEM((CC, D), jnp.float32),
            pltpu.VMEM((CC, D), jnp.float32),
            pltpu.SemaphoreType.DMA,
            pltpu.SemaphoreType.DMA,
        ],
        compiler_params=pltpu.CompilerParams(
            needs_layout_passes=False, use_tc_tiling_on_sc=False),
    )(_sc_contrib_body)


def _sc_contrib_body(xw_hbm, row_hbm, col_hbm, ew_hbm, dis_hbm, out_hbm,
                     rowv, colv, ewv, normv, disv, bufa, bufb, sema, semb):
    cid = lax.axis_index("c")
    sid = lax.axis_index("s")
    wid = sid * NC + cid
    base = wid * WSLOT
    pltpu.sync_copy(row_hbm.at[pl.ds(base, WSLOT)], rowv)
    pltpu.sync_copy(col_hbm.at[pl.ds(base, WSLOT)], colv)
    pltpu.sync_copy(ew_hbm.at[pl.ds(base, WSLOT)], ewv)
    pltpu.sync_copy(dis_hbm, disv)

    def norm_group(g, _):
        r16 = rowv[pl.ds(g * 16, 16)]
        c16 = colv[pl.ds(g * 16, 16)]
        w16 = ewv[pl.ds(g * 16, 16)]
        dr = plsc.load_gather(disv, [r16])
        dc = plsc.load_gather(disv, [c16])
        normv[pl.ds(g * 16, 16)] = (dr * w16) * dc
        return 0

    lax.fori_loop(0, WSLOT // 16, norm_group, 0)

    nch = jnp.where(wid == NW - 1, 84, 86)

    def gather(ch, buf, sem):
        pltpu.async_copy(xw_hbm.at[rowv.at[pl.ds(ch * CC, CC)]], buf, sem)

    def wait(buf, sem):
        pltpu.make_async_copy(xw_hbm.at[rowv.at[pl.ds(0, CC)]], buf, sem).wait()

    def scale_store(ch, buf):
        def body(i, _):
            sc = plsc.load_gather(normv, [jnp.full((16,), ch * CC + i, jnp.int32)])
            for j in range(D // 16):
                buf[i, pl.ds(j * 16, 16)] = buf[i, pl.ds(j * 16, 16)] * sc
            return 0

        lax.fori_loop(0, CC, body, 0)
        pltpu.sync_copy(buf, out_hbm.at[pl.ds(base + ch * CC, CC)])

    gather(0, bufa, sema)

    def outer2(t, _):
        ch = 2 * t
        gather(ch + 1, bufb, semb)
        wait(bufa, sema)
        scale_store(ch, bufa)

        @pl.when(ch + 2 < nch)
        def _():
            gather(ch + 2, bufa, sema)

        wait(bufb, semb)
        scale_store(ch + 1, bufb)
        return 0

    lax.fori_loop(0, nch // 2, outer2, 0)


def _make_sc_agg():
    mesh = plsc.VectorSubcoreMesh(
        core_axis_name="c", subcore_axis_name="s",
        num_cores=NC, num_subcores=NS)
    return functools.partial(
        pl.kernel,
        out_type=jax.ShapeDtypeStruct((NC, 2, NPAD, DH), jnp.float32),
        mesh=mesh,
        scratch_types=[
            pltpu.VMEM((CH, C), jnp.int32),
            pltpu.VMEM((CH, C), jnp.int32),
            pltpu.VMEM((CH * C,), jnp.float32),
            pltpu.VMEM((C, DH), jnp.float32),
            pltpu.VMEM((C, DH), jnp.float32),
            pltpu.VMEM((64, DH), jnp.float32),
            pltpu.VMEM_SHARED((NPAD, DH), jnp.float32),
            pltpu.SemaphoreType.DMA,
            pltpu.SemaphoreType.DMA,
        ],
        compiler_params=pltpu.CompilerParams(
            needs_layout_passes=False, use_tc_tiling_on_sc=False),
    )(_sc_agg_body)


def _sc_agg_body(y0_hbm, y1_hbm, row_hbm, col_hbm, ev_hbm, out_hbm,
                 rowv, colv, evv, bufa, bufb, zbuf, acc, sema, semb):
    cid = lax.axis_index("c")
    sid = lax.axis_index("s")
    wid = sid * NC + cid
    pltpu.sync_copy(row_hbm.at[wid], rowv)
    pltpu.sync_copy(col_hbm.at[wid], colv)
    pltpu.sync_copy(ev_hbm.at[wid], evv)

    zero = jnp.zeros((16,), jnp.float32)

    def zrow(i, _):
        for j in range(DH // 16):
            zbuf[i, pl.ds(j * 16, 16)] = zero
        return 0

    lax.fori_loop(0, 64, zrow, 0)

    def scale_scatter(ch, buf):
        def body(i, _):
            sc = plsc.load_gather(evv, [jnp.full((16,), ch * C + i, jnp.int32)])
            for j in range(DH // 16):
                buf[i, pl.ds(j * 16, 16)] = buf[i, pl.ds(j * 16, 16)] * sc
            return 0

        lax.fori_loop(0, C, body, 0)
        pltpu.sync_copy(buf, acc.at[colv.at[ch]], add=True)

    for half, y_hbm in ((0, y0_hbm), (1, y1_hbm)):
        for k in range(ZB // 64):
            pltpu.sync_copy(zbuf, acc.at[pl.ds(sid * ZB + k * 64, 64)])
        plsc.subcore_barrier()

        def gather(ch, buf, sem):
            pltpu.async_copy(y_hbm.at[rowv.at[ch]], buf, sem)

        def wait(buf, sem):
            pltpu.make_async_copy(y_hbm.at[rowv.at[0]], buf, sem).wait()

        gather(0, bufa, sema)

        def outer(t, _):
            ch = 2 * t
            gather(ch + 1, bufb, semb)
            wait(bufa, sema)
            scale_scatter(ch, bufa)

            @pl.when(ch + 2 < CH)
            def _():
                gather(ch + 2, bufa, sema)

            wait(bufb, semb)
            scale_scatter(ch + 1, bufb)
            return 0

        lax.fori_loop(0, CH // 2, outer, 0)
        plsc.subcore_barrier()
        pltpu.sync_copy(acc.at[pl.ds(sid * ZB, ZB)],
                        out_hbm.at[cid, half, pl.ds(sid * ZB, ZB)])
        plsc.subcore_barrier()


_SC_CACHE = {}


def _sc_agg(y0, y1, row3, col3, ev2):
    if "agg" not in _SC_CACHE:
        _SC_CACHE["agg"] = _make_sc_agg()
    return _SC_CACHE["agg"](y0, y1, row3, col3, ev2)


def _sc_contrib(xw, row_f, col_f, ew_f, dis):
    if "contrib" not in _SC_CACHE:
        _SC_CACHE["contrib"] = _make_sc_contrib()
    return _SC_CACHE["contrib"](xw, row_f, col_f, ew_f, dis)


_MB = 2000


def _mm_raw_body(x_ref, w_ref, o_ref):
    o_ref[...] = jnp.dot(x_ref[...], w_ref[...],
                         preferred_element_type=jnp.float32)


def _tc_mm_raw(x, w):
    return pl.pallas_call(
        _mm_raw_body,
        grid=(N // _MB,),
        in_specs=[
            pl.BlockSpec((_MB, D), lambda i: (i, 0)),
            pl.BlockSpec((D, D), lambda i: (0, 0)),
        ],
        out_specs=pl.BlockSpec((_MB, D), lambda i: (i, 0)),
        out_shape=jax.ShapeDtypeStruct((N, D), jnp.float32),
    )(x, w)


def _mm2_body(x_ref, w_ref, dis_ref, y0_ref, y1_ref):
    acc = jnp.dot(x_ref[...], w_ref[...], preferred_element_type=jnp.float32)
    y = dis_ref[...] * acc
    y0_ref[...] = y[:, :DH]
    y1_ref[...] = y[:, DH:]


def _tc_mm2(x, w, dis):
    return pl.pallas_call(
        _mm2_body,
        grid=(N // _MB,),
        in_specs=[
            pl.BlockSpec((_MB, D), lambda i: (i, 0)),
            pl.BlockSpec((D, D), lambda i: (0, 0)),
            pl.BlockSpec((_MB, 1), lambda i: (i, 0)),
        ],
        out_specs=[
            pl.BlockSpec((_MB, DH), lambda i: (i, 0)),
            pl.BlockSpec((_MB, DH), lambda i: (i, 0)),
        ],
        out_shape=[
            jax.ShapeDtypeStruct((N, DH), jnp.float32),
            jax.ShapeDtypeStruct((N, DH), jnp.float32),
        ],
    )(x, w, dis)


def _epi_body(a00_ref, a01_ref, a10_ref, a11_ref, y0_ref, y1_ref, dis_ref,
              h_ref):
    dis = dis_ref[...]
    h_ref[:, :DH] = jnp.maximum(
        dis * (a00_ref[...] + a10_ref[...] + y0_ref[...]), 0.0)
    h_ref[:, DH:] = jnp.maximum(
        dis * (a01_ref[...] + a11_ref[...] + y1_ref[...]), 0.0)


def _tc_epi(a, y0, y1, dis):
    return pl.pallas_call(
        _epi_body,
        grid=(N // _MB,),
        in_specs=[
            pl.BlockSpec((_MB, DH), lambda i: (i, 0)),
            pl.BlockSpec((_MB, DH), lambda i: (i, 0)),
            pl.BlockSpec((_MB, DH), lambda i: (i, 0)),
            pl.BlockSpec((_MB, DH), lambda i: (i, 0)),
            pl.BlockSpec((_MB, DH), lambda i: (i, 0)),
            pl.BlockSpec((_MB, DH), lambda i: (i, 0)),
            pl.BlockSpec((_MB, 1), lambda i: (i, 0)),
        ],
        out_specs=pl.BlockSpec((_MB, D), lambda i: (i, 0)),
        out_shape=jax.ShapeDtypeStruct((N, D), jnp.float32),
    )(a[0, 0], a[0, 1], a[1, 0], a[1, 1], y0, y1, dis)


def _gru_cell(xi, h, Wih, Whh, bih, bhh):
    gi = xi @ Wih.T + bih
    gh = h @ Whh.T + bhh
    ir, iz, inn = jnp.split(gi, 3, axis=-1)
    hr, hz, hn = jnp.split(gh, 3, axis=-1)
    r = jax.nn.sigmoid(ir + hr)
    z = jax.nn.sigmoid(iz + hz)
    n = jnp.tanh(inn + r * hn)
    return (1.0 - z) * n + z * h


def _layer_weight(xin, score, Wih, Whh, bih, bhh, Winit):
    vals, idx = jax.lax.top_k(score, D)
    xt = xin[idx] * vals[:, None]
    return _gru_cell(xt, Winit, Wih, Whh, bih, bhh)


def kernel(x, edge_index, edge_weight, p1, Wih1, Whh1, bih1, bhh1, Winit1,
           p2, Wih2, Whh2, bih2, bhh2, Winit2, lin1_W, lin1_b, lin2_W, lin2_b):
    row, col = edge_index[0], edge_index[1]
    keep = (row != col).astype(edge_weight.dtype)
    ev = edge_weight * keep

    loop = jnp.arange(N, dtype=row.dtype)
    row_f = jnp.concatenate([row, loop])
    col_f = jnp.concatenate([col, loop])
    ew_f = jnp.concatenate([ev, jnp.ones((N,), dtype=edge_weight.dtype)])
    deg = jax.ops.segment_sum(ew_f, col_f, num_segments=N)
    safe = jnp.where(deg > 0, deg, 1.0)
    dis = jnp.where(deg > 0, 1.0 / jnp.sqrt(safe), 0.0)
    norm = dis[row_f] * ew_f * dis[col_f]

    score1 = jnp.tanh((x @ p1) / (jnp.linalg.norm(p1) + 1e-16))
    W1 = _layer_weight(x, score1, Wih1, Whh1, bih1, bhh1, Winit1)
    xw1 = _tc_mm_raw(x, W1)
    pad = jnp.zeros((EFPAD - EF,), dtype=row_f.dtype)
    row_fp = jnp.concatenate([row_f, pad])
    col_fp = jnp.concatenate([col_f, pad])
    ew_fp = jnp.concatenate([ew_f, jnp.zeros((EFPAD - EF,), ew_f.dtype)])
    contrib = _sc_contrib(xw1, row_fp, col_fp, ew_fp, dis)
    h = jax.nn.relu(jax.ops.segment_sum(contrib, col_f, num_segments=N))

    score2 = jnp.tanh((h @ p2) / (jnp.linalg.norm(p2) + 1e-16))
    W2 = _layer_weight(h, score2, Wih2, Whh2, bih2, bhh2, Winit2)

    dis2d = dis[:, None]
    y20, y21 = _tc_mm2(h, W2, dis2d)

    row3 = row.reshape(NW, CH, C)
    col3 = col.reshape(NW, CH, C)
    ev2 = ev.reshape(NW, CH * C)
    a2 = _sc_agg(y20, y21, row3, col3, ev2)[:, :, :N]
    h2 = _tc_epi(a2, y20, y21, dis2d)

    h2 = h2 @ lin1_W.T + lin1_b
    h2 = h2 @ lin2_W.T + lin2_b
    return h2
